# Initial kernel scaffold; baseline (speedup 1.0000x reference)
import jax
import jax.numpy as jnp
from jax.experimental import pallas as pl


def kernel(x, edge_index, W_fwd, att_src_fwd, att_dst_fwd, b_fwd, W_bwd, att_src_bwd, att_dst_bwd, b_bwd, W_fuse, b_fuse, gamma, beta):
    raise NotImplementedError("write your pallas kernel here")


# speedup vs baseline: 28.3653x; 28.3653x over previous
import functools

import jax
import jax.numpy as jnp
from jax import lax
from jax.experimental import pallas as pl
from jax.experimental.pallas import tpu as pltpu
from jax.experimental.pallas import tpu_sc as plsc

NCORES = 2
NTILES = 16
LANES = 16
ROWW = 80
CHUNK = 128


def _leaky(v):
    return jnp.where(v >= 0, v, 0.2 * v)


def _k1_body(x_ref, w_ref, att_ref, h_ref, a_ref):
    x = x_ref[...]
    h = jnp.dot(x, w_ref[0], preferred_element_type=jnp.float32)
    br = h.shape[0]
    ones = jnp.ones((br, 1), jnp.float32)
    zeros = jnp.zeros((br, ROWW - h.shape[1] - 1), jnp.float32)
    h_ref[...] = jnp.concatenate([h, ones, zeros], axis=1)
    a_s = jnp.dot(h, att_ref[0, 0][:, None], preferred_element_type=jnp.float32)
    a_d = jnp.dot(h, att_ref[0, 1][:, None], preferred_element_type=jnp.float32)
    a_ref[...] = jnp.concatenate([a_s, a_d], axis=1)


def _k1(x, wcat, attcat, n, d, f, br):
    nb = n // br
    return pl.pallas_call(
        _k1_body,
        grid=(2, nb),
        in_specs=[
            pl.BlockSpec((br, d), lambda c, b: (b, 0)),
            pl.BlockSpec((1, d, f), lambda c, b: (c, 0, 0)),
            pl.BlockSpec((1, 2, f), lambda c, b: (c, 0, 0)),
        ],
        out_specs=[
            pl.BlockSpec((br, ROWW), lambda c, b: (c * nb + b, 0)),
            pl.BlockSpec((br, 2), lambda c, b: (c * nb + b, 0)),
        ],
        out_shape=[
            jax.ShapeDtypeStruct((2 * n, ROWW), jnp.float32),
            jax.ShapeDtypeStruct((2 * n, 2), jnp.float32),
        ],
    )(x, wcat, attcat)


def _k2_body(n, hpad_ref, a_ref, init_ref, bounds_ref):
    a_s = a_ref[:, 0:1]
    a_d = a_ref[:, 1:2]
    bf = _leaky(jnp.max(a_s[:n]) + jnp.max(a_d[:n]))
    bb = _leaky(jnp.max(a_s[n:]) + jnp.max(a_d[n:]))
    rows = lax.broadcasted_iota(jnp.int32, a_s.shape, 0)
    bvec = jnp.where(rows < n, bf, bb)
    ee = jnp.exp(_leaky(a_s + a_d) - bvec)
    init_ref[...] = hpad_ref[...] * ee
    bounds_ref[...] = jnp.concatenate(
        [jnp.full((1, 16), bf, jnp.float32), jnp.full((1, 16), bb, jnp.float32)])


def _k2(hpad, a, n):
    return pl.pallas_call(
        functools.partial(_k2_body, n),
        out_shape=[
            jax.ShapeDtypeStruct((2 * n, ROWW), jnp.float32),
            jax.ShapeDtypeStruct((2, 16), jnp.float32),
        ],
    )(hpad, a)


def _sc_edge_kernel(n, e, ept_pad):
    epad = NTILES * ept_pad
    nblk = n // 8
    nchunks = ept_pad // CHUNK
    mesh = plsc.VectorSubcoreMesh(core_axis_name="c", subcore_axis_name="s",
                                  num_cores=NCORES, num_subcores=NTILES)

    @functools.partial(
        pl.kernel,
        out_type=jax.ShapeDtypeStruct((2 * n, ROWW), jnp.float32),
        mesh=mesh,
        compiler_params=pltpu.CompilerParams(needs_layout_passes=False,
                                             use_tc_tiling_on_sc=False),
        scratch_types=[
            pltpu.VMEM((n,), jnp.float32),
            pltpu.VMEM((n,), jnp.float32),
            pltpu.VMEM((16,), jnp.float32),
            pltpu.VMEM((CHUNK,), jnp.int32),
            pltpu.VMEM((CHUNK,), jnp.int32),
            pltpu.VMEM((CHUNK,), jnp.int32),
            pltpu.VMEM((CHUNK,), jnp.float32),
            pltpu.VMEM((CHUNK, ROWW), jnp.float32),
            pltpu.VMEM_SHARED((n, ROWW), jnp.float32),
            pltpu.SemaphoreType.DMA,
        ],
    )
    def body(eidx, asrc, adst, bounds, hpad, init, accum,
             asrc_t, adst_t, bnd_t, gidx_t, sidx_t, hidx_t, eexp_t, rows_t,
             acc_sh, sem):
        c = lax.axis_index("c")
        s = lax.axis_index("s")
        coff = pl.multiple_of(c * n, 8)
        pltpu.sync_copy(asrc.at[pl.ds(coff, n)], asrc_t)
        pltpu.sync_copy(adst.at[pl.ds(coff, n)], adst_t)
        pltpu.sync_copy(bounds.at[pl.ds(c * 16, 16)], bnd_t)
        nmine = nblk // NTILES + jnp.where(s < nblk % NTILES, 1, 0)

        def init_body(k, _):
            t = (s + k * NTILES) * 8
            pltpu.sync_copy(init.at[pl.ds(c * n + t, 8)],
                            acc_sh.at[pl.ds(t, 8)])
            return 0

        lax.fori_loop(0, nmine, init_body, 0)
        plsc.subcore_barrier()

        def chunk_body(k, _):
            base = s * ept_pad + k * CHUNK
            goff = pl.multiple_of(c * epad + base, 8)
            soff = pl.multiple_of((1 - c) * epad + base, 8)
            pltpu.sync_copy(eidx.at[pl.ds(goff, CHUNK)], gidx_t)
            pltpu.sync_copy(eidx.at[pl.ds(soff, CHUNK)], sidx_t)
            bv = bnd_t[...]

            def vec_body(j, _):
                gi = gidx_t[pl.ds(j * LANES, LANES)]
                si = sidx_t[pl.ds(j * LANES, LANES)]
                av = plsc.load_gather(asrc_t, [gi])
                ad = plsc.load_gather(adst_t, [si])
                ev = _leaky(av + ad)
                ee = jnp.exp(ev - bv)
                ids = base + j * LANES + lax.iota(jnp.int32, LANES)
                ee = jnp.where(ids < e, ee, 0.0)
                eexp_t[pl.ds(j * LANES, LANES)] = ee
                hidx_t[pl.ds(j * LANES, LANES)] = gi + c * n
                return 0

            lax.fori_loop(0, CHUNK // LANES, vec_body, 0)
            pltpu.async_copy(hpad.at[hidx_t], rows_t, sem).wait()

            def scale_body(jj, _):
                ee16 = eexp_t[pl.ds(jj * LANES, LANES)]
                for i in range(LANES):
                    sv = ee16[i]
                    r = jj * LANES + i
                    for q in range(ROWW // LANES):
                        sl = pl.ds(q * LANES, LANES)
                        rows_t[r, sl] = rows_t[r, sl] * sv
                return 0

            lax.fori_loop(0, CHUNK // LANES, scale_body, 0)
            pltpu.sync_copy(rows_t, acc_sh.at[sidx_t], add=True)
            return 0

        lax.fori_loop(0, nchunks, chunk_body, 0)
        plsc.subcore_barrier()

        def out_body(k, _):
            t = (s + k * NTILES) * 8
            pltpu.sync_copy(acc_sh.at[pl.ds(t, 8)],
                            accum.at[pl.ds(c * n + t, 8)])
            return 0

        lax.fori_loop(0, nmine, out_body, 0)

    return body


def _k3_body(n, f, acc_ref, bf_ref, bb_ref, wf_ref, bfu_ref, g_ref, be_ref, out_ref):
    den = acc_ref[:, f:f + 1]
    agg = acc_ref[:, :f] / den
    outf = agg[:n] + bf_ref[...][None, :]
    outb = agg[n:] + bb_ref[...][None, :]
    combined = jnp.concatenate([outf, outb], axis=1)
    fused = jnp.dot(combined, wf_ref[...], preferred_element_type=jnp.float32)
    fused = fused + bfu_ref[...][None, :]
    mu = jnp.mean(fused, axis=0, keepdims=True)
    var = jnp.mean((fused - mu) ** 2, axis=0, keepdims=True)
    normed = (fused - mu) / jnp.sqrt(var + 1e-5) * g_ref[...][None, :] + be_ref[...][None, :]
    out_ref[...] = jnp.maximum(normed, 0.0)


def _k3(accum, b_fwd, b_bwd, w_fuse, b_fuse, gamma, beta, n, f):
    hid = w_fuse.shape[0]
    return pl.pallas_call(
        functools.partial(_k3_body, n, f),
        out_shape=jax.ShapeDtypeStruct((n, hid), jnp.float32),
    )(accum, b_fwd, b_bwd, w_fuse, b_fuse, gamma, beta)


def kernel(x, edge_index, W_fwd, att_src_fwd, att_dst_fwd, b_fwd,
           W_bwd, att_src_bwd, att_dst_bwd, b_bwd, W_fuse, b_fuse, gamma, beta):
    n, d = x.shape
    f = W_fwd.shape[1]
    e = edge_index.shape[1]

    wcat = jnp.stack([W_fwd, W_bwd])
    attcat = jnp.stack([jnp.stack([att_src_fwd, att_dst_fwd]),
                        jnp.stack([att_src_bwd, att_dst_bwd])])

    br = 1000 if n % 1000 == 0 else 8
    hpad, a = _k1(x, wcat, attcat, n, d, f, br)
    init, bounds = _k2(hpad, a, n)
    asrc = a[:, 0] + 0.0
    adst = a[:, 1] + 0.0

    ept_pad = -(-e // (NTILES * CHUNK)) * CHUNK
    epad = NTILES * ept_pad
    src = edge_index[0].astype(jnp.int32)
    dst = edge_index[1].astype(jnp.int32)
    pad = jnp.zeros((epad - e,), jnp.int32)
    eidx = jnp.concatenate([src, pad, dst, pad])

    sc = _sc_edge_kernel(n, e, ept_pad)
    accum = sc(eidx, asrc, adst, bounds.reshape(-1), hpad, init)

    return _k3(accum, b_fwd, b_bwd, W_fuse, b_fuse, gamma, beta, n, f)


# speedup vs baseline: 39.3938x; 1.3888x over previous
import functools

import jax
import jax.numpy as jnp
from jax import lax
from jax.experimental import pallas as pl
from jax.experimental.pallas import tpu as pltpu
from jax.experimental.pallas import tpu_sc as plsc

NCORES = 2
NTILES = 16
LANES = 16
CHUNK = 128


def _leaky(v):
    return jnp.where(v >= 0, v, 0.2 * v)


def _k1_body(x_ref, w_ref, att_ref, h_ref, a_ref):
    x = x_ref[...]
    h = jnp.dot(x, w_ref[0], preferred_element_type=jnp.float32)
    h_ref[...] = h
    a_s = jnp.dot(h, att_ref[0, 0][:, None], preferred_element_type=jnp.float32)
    a_d = jnp.dot(h, att_ref[0, 1][:, None], preferred_element_type=jnp.float32)
    a_ref[...] = jnp.concatenate([a_s, a_d], axis=1)


def _k1(x, wcat, attcat, n, d, f, br):
    nb = n // br
    return pl.pallas_call(
        _k1_body,
        grid=(2, nb),
        in_specs=[
            pl.BlockSpec((br, d), lambda c, b: (b, 0)),
            pl.BlockSpec((1, d, f), lambda c, b: (c, 0, 0)),
            pl.BlockSpec((1, 2, f), lambda c, b: (c, 0, 0)),
        ],
        out_specs=[
            pl.BlockSpec((br, f), lambda c, b: (c * nb + b, 0)),
            pl.BlockSpec((br, 2), lambda c, b: (c * nb + b, 0)),
        ],
        out_shape=[
            jax.ShapeDtypeStruct((2 * n, f), jnp.float32),
            jax.ShapeDtypeStruct((2 * n, 2), jnp.float32),
        ],
    )(x, wcat, attcat)


def _k2_body(n, h_ref, a_ref, init_ref, self_ref, bounds_ref):
    a_s = a_ref[:, 0:1]
    a_d = a_ref[:, 1:2]
    bf = _leaky(jnp.max(a_s[:n]) + jnp.max(a_d[:n]))
    bb = _leaky(jnp.max(a_s[n:]) + jnp.max(a_d[n:]))
    rows = lax.broadcasted_iota(jnp.int32, a_s.shape, 0)
    bvec = jnp.where(rows < n, bf, bb)
    ee = jnp.exp(_leaky(a_s + a_d) - bvec)
    init_ref[...] = h_ref[...] * ee
    self_ref[...] = ee
    bounds_ref[...] = jnp.concatenate(
        [jnp.full((1, 16), bf, jnp.float32), jnp.full((1, 16), bb, jnp.float32)])


def _k2(h, a, n, f):
    return pl.pallas_call(
        functools.partial(_k2_body, n),
        out_shape=[
            jax.ShapeDtypeStruct((2 * n, f), jnp.float32),
            jax.ShapeDtypeStruct((2 * n, 1), jnp.float32),
            jax.ShapeDtypeStruct((2, 16), jnp.float32),
        ],
    )(h, a)


def _sc_edge_kernel(n, e, f, ept_pad, npad):
    nchunks = ept_pad // CHUNK
    npairs = nchunks // 2
    nblk = n // 8
    nppt = npad // NTILES
    mesh = plsc.VectorSubcoreMesh(core_axis_name="c", subcore_axis_name="s",
                                  num_cores=NCORES, num_subcores=NTILES)

    @functools.partial(
        pl.kernel,
        out_type=[jax.ShapeDtypeStruct((2 * n, f), jnp.float32),
                  jax.ShapeDtypeStruct((2 * npad,), jnp.float32)],
        mesh=mesh,
        compiler_params=pltpu.CompilerParams(needs_layout_passes=False,
                                             use_tc_tiling_on_sc=False),
        scratch_types=[
            pltpu.VMEM((n,), jnp.float32),
            pltpu.VMEM((n,), jnp.float32),
            pltpu.VMEM((16,), jnp.float32),
            pltpu.VMEM((2, CHUNK), jnp.int32),
            pltpu.VMEM((4, CHUNK), jnp.int32),
            pltpu.VMEM((2, CHUNK), jnp.int32),
            pltpu.VMEM((2, CHUNK), jnp.float32),
            pltpu.VMEM((npad,), jnp.float32),
            pltpu.VMEM((NTILES, nppt), jnp.float32),
            pltpu.VMEM((CHUNK, 64), jnp.float32),
            pltpu.VMEM((CHUNK, 64), jnp.float32),
            pltpu.VMEM_SHARED((n, 64), jnp.float32),
            pltpu.VMEM_SHARED((NTILES, npad), jnp.float32),
            pltpu.SemaphoreType.DMA,
            pltpu.SemaphoreType.DMA,
            pltpu.SemaphoreType.DMA,
            pltpu.SemaphoreType.DMA,
            pltpu.SemaphoreType.DMA,
            pltpu.SemaphoreType.DMA,
        ],
    )
    def body(eidx, asrc, adst, bounds, h_hbm, init, num, den,
             asrc_t, adst_t, bnd_t, gidx_t, sidx_t, hidx_t, eexp_t, den_t,
             red_buf, rowsA, rowsB, acc_sh, den_all_sh,
             sem_ia, sem_ib, sem_ga, sem_gb, sem_sa, sem_sb):
        c = lax.axis_index("c")
        s = lax.axis_index("s")
        coff = pl.multiple_of(c * n, 8)
        pltpu.sync_copy(asrc.at[pl.ds(coff, n)], asrc_t)
        pltpu.sync_copy(adst.at[pl.ds(coff, n)], adst_t)
        pltpu.sync_copy(bounds.at[pl.ds(c * 16, 16)], bnd_t)

        zero16 = jnp.zeros((LANES,), jnp.float32)

        def z_body(i, _):
            den_t[pl.ds(i * LANES, LANES)] = zero16
            return 0

        lax.fori_loop(0, npad // LANES, z_body, 0)

        nmine = nblk // NTILES + jnp.where(s < nblk % NTILES, 1, 0)

        def init_body(k, _):
            t = (s + k * NTILES) * 8
            pltpu.sync_copy(init.at[pl.ds(c * n + t, 8)], acc_sh.at[pl.ds(t, 8)])
            return 0

        lax.fori_loop(0, nmine, init_body, 0)
        plsc.subcore_barrier()

        bv = bnd_t[...]
        ebase = s * ept_pad

        def i_start(kc, b, slot):
            pltpu.async_copy(eidx.at[c, s, kc], gidx_t.at[b],
                             sem_ia if b == 0 else sem_ib)
            pltpu.async_copy(eidx.at[1 - c, s, kc], sidx_t.at[slot],
                             sem_ia if b == 0 else sem_ib)

        def i_wait(kc, b, slot):
            sem = sem_ia if b == 0 else sem_ib
            pltpu.make_async_copy(eidx.at[c, s, kc], gidx_t.at[b], sem).wait()
            pltpu.make_async_copy(eidx.at[1 - c, s, kc], sidx_t.at[slot],
                                  sem).wait()

        def compute(kc, b, slot):
            def cv(j, _):
                sl = pl.ds(j * LANES, LANES)
                gi = gidx_t[b, sl]
                si = sidx_t[slot, sl]
                av = plsc.load_gather(asrc_t, [gi])
                ad = plsc.load_gather(adst_t, [si])
                ee = jnp.exp(_leaky(av + ad) - bv)
                ids = ebase + kc * CHUNK + j * LANES + lax.iota(jnp.int32, LANES)
                ee = jnp.where(ids < e, ee, 0.0)
                eexp_t[b, sl] = ee
                hidx_t[b, sl] = gi + coff
                plsc.addupdate_scatter(den_t, [si], ee)
                return 0

            lax.fori_loop(0, CHUNK // LANES, cv, 0)

        def g_start(b, rows_ref, sem):
            pltpu.async_copy(h_hbm.at[hidx_t.at[b]], rows_ref, sem)

        def g_wait(b, rows_ref, sem):
            pltpu.make_async_copy(h_hbm.at[hidx_t.at[b]], rows_ref, sem).wait()

        def s_start(slot, rows_ref, sem):
            pltpu.async_copy(rows_ref, acc_sh.at[sidx_t.at[slot]], sem, add=True)

        def s_wait(slot, rows_ref, sem):
            pltpu.make_async_copy(rows_ref, acc_sh.at[sidx_t.at[slot]],
                                  sem).wait()

        def scale(b, rows_ref):
            def sb(jj, _):
                ee16 = eexp_t[b, pl.ds(jj * LANES, LANES)]
                for i in range(LANES):
                    sv = ee16[i]
                    r = jj * LANES + i
                    for q in range(64 // LANES):
                        sl = pl.ds(q * LANES, LANES)
                        rows_ref[r, sl] = rows_ref[r, sl] * sv
                return 0

            lax.fori_loop(0, CHUNK // LANES, sb, 0)

        i_start(0, 0, 0)
        i_start(1, 1, 1)

        def pair_body(p, _):
            k0 = 2 * p
            k1 = k0 + 1
            pm = p % 2
            slot0 = 2 * pm
            slot1 = slot0 + 1
            i_wait(k0, 0, slot0)
            compute(k0, 0, slot0)
            g_start(0, rowsA, sem_ga)
            i_wait(k1, 1, slot1)
            compute(k1, 1, slot1)

            @pl.when(p > 0)
            def _():
                s_wait(3 - slot0, rowsB, sem_sb)

            g_start(1, rowsB, sem_gb)

            @pl.when(p < npairs - 1)
            def _():
                i_start(k0 + 2, 0, 2 - slot0)
                i_start(k1 + 2, 1, 3 - slot0)

            g_wait(0, rowsA, sem_ga)
            scale(0, rowsA)
            s_start(slot0, rowsA, sem_sa)
            g_wait(1, rowsB, sem_gb)
            scale(1, rowsB)
            s_wait(slot0, rowsA, sem_sa)
            s_start(slot1, rowsB, sem_sb)
            return 0

        lax.fori_loop(0, npairs, pair_body, 0)
        s_wait(2 * ((npairs - 1) % 2) + 1, rowsB, sem_sb)
        plsc.subcore_barrier()

        pltpu.sync_copy(den_t, den_all_sh.at[s])
        plsc.subcore_barrier()
        dcol = pl.multiple_of(s * nppt, 8)
        pltpu.sync_copy(den_all_sh.at[:, pl.ds(dcol, nppt)], red_buf)

        def dred_body(j, _):
            sl = pl.ds(j * LANES, LANES)
            v = red_buf[0, sl]
            for r in range(1, NTILES):
                v = v + red_buf[r, sl]
            den_t[sl] = v
            return 0

        lax.fori_loop(0, nppt // LANES, dred_body, 0)
        pltpu.sync_copy(den_t.at[pl.ds(0, nppt)],
                        den.at[pl.ds(c * npad + dcol, nppt)])

        def out_body(k, _):
            t = (s + k * NTILES) * 8
            pltpu.sync_copy(acc_sh.at[pl.ds(t, 8)], num.at[pl.ds(c * n + t, 8)])
            return 0

        lax.fori_loop(0, nmine, out_body, 0)

    return body


def _k3_body(n, f, num_ref, dene_ref, selfee_ref, bf_ref, bb_ref, wf_ref,
             bfu_ref, g_ref, be_ref, out_ref):
    den = dene_ref[...] + selfee_ref[...]
    agg = num_ref[...] / den
    outf = agg[:n] + bf_ref[...][None, :]
    outb = agg[n:] + bb_ref[...][None, :]
    combined = jnp.concatenate([outf, outb], axis=1)
    fused = jnp.dot(combined, wf_ref[...], preferred_element_type=jnp.float32)
    fused = fused + bfu_ref[...][None, :]
    mu = jnp.mean(fused, axis=0, keepdims=True)
    var = jnp.mean((fused - mu) ** 2, axis=0, keepdims=True)
    normed = (fused - mu) / jnp.sqrt(var + 1e-5) * g_ref[...][None, :] + be_ref[...][None, :]
    out_ref[...] = jnp.maximum(normed, 0.0)


def _k3(num, dene, selfee, b_fwd, b_bwd, w_fuse, b_fuse, gamma, beta, n, f):
    hid = w_fuse.shape[0]
    return pl.pallas_call(
        functools.partial(_k3_body, n, f),
        out_shape=jax.ShapeDtypeStruct((n, hid), jnp.float32),
    )(num, dene, selfee, b_fwd, b_bwd, w_fuse, b_fuse, gamma, beta)


def kernel(x, edge_index, W_fwd, att_src_fwd, att_dst_fwd, b_fwd,
           W_bwd, att_src_bwd, att_dst_bwd, b_bwd, W_fuse, b_fuse, gamma, beta):
    n, d = x.shape
    f = W_fwd.shape[1]
    e = edge_index.shape[1]

    wcat = jnp.stack([W_fwd, W_bwd])
    attcat = jnp.stack([jnp.stack([att_src_fwd, att_dst_fwd]),
                        jnp.stack([att_src_bwd, att_dst_bwd])])

    br = 1000 if n % 1000 == 0 else 8
    h, a = _k1(x, wcat, attcat, n, d, f, br)
    init, selfee, bounds = _k2(h, a, n, f)
    asrc = a[:, 0] + 0.0
    adst = a[:, 1] + 0.0

    ept_pad = -(-e // (NTILES * 2 * CHUNK)) * 2 * CHUNK
    epad = NTILES * ept_pad
    nchunks = ept_pad // CHUNK
    npad = -(-n // (NTILES * LANES)) * NTILES * LANES
    src = edge_index[0].astype(jnp.int32)
    dst = edge_index[1].astype(jnp.int32)
    pad = jnp.zeros((epad - e,), jnp.int32)
    eidx = jnp.stack([jnp.concatenate([src, pad]),
                      jnp.concatenate([dst, pad])])
    eidx = eidx.reshape(2, NTILES, nchunks, CHUNK)

    sc = _sc_edge_kernel(n, e, f, ept_pad, npad)
    num, den = sc(eidx, asrc, adst, bounds.reshape(-1), h, init)
    dene = den.reshape(2, npad)[:, :n].reshape(2 * n, 1)

    return _k3(num, dene, selfee, b_fwd, b_bwd, W_fuse, b_fuse, gamma, beta, n, f)
